# baseline (speedup 1.0000x reference)
import functools

import jax
import jax.numpy as jnp
from jax import lax
from jax.experimental import pallas as pl
from jax.experimental.pallas import tpu as pltpu
from jax.experimental.pallas import tpu_sc as plsc

MAX_LEN = 8192

_NC = 2
_NS = 16
_NW = _NC * _NS
_CH = 16


def _sc_body(bsz, seq_len, d, x_hbm, t_hbm, o_hbm, xv, tv):
    wid = lax.axis_index("c") * _NS + lax.axis_index("s")
    seq_per_w = seq_len // _NW
    seq_base = wid * seq_per_w
    groups_per_lane = d // 16

    def chunk(g, carry):
        t0 = seq_base + g * _CH
        pltpu.sync_copy(t_hbm.at[pl.ds(t0, _CH), :], tv)
        for b in range(bsz):
            r0 = b * seq_len + t0
            pltpu.sync_copy(x_hbm.at[pl.ds(r0, _CH), :], xv)

            def row(i, c2):
                for j in range(groups_per_lane):
                    sl = pl.ds(j * 16, 16)
                    xv[i, sl] = xv[i, sl] + tv[i, sl]
                return c2

            lax.fori_loop(0, _CH, row, 0)
            pltpu.sync_copy(xv, o_hbm.at[pl.ds(r0, _CH), :])
        return carry

    lax.fori_loop(0, seq_per_w // _CH, chunk, 0)


def kernel(x, table):
    bsz, seq_len, d = x.shape
    if seq_len > MAX_LEN:
        x = x[:, -MAX_LEN:, :]
        seq_len = MAX_LEN
    x2 = x.reshape(bsz * seq_len, d)
    mesh = plsc.VectorSubcoreMesh(core_axis_name="c", subcore_axis_name="s")
    body = functools.partial(_sc_body, bsz, seq_len, d)
    out = pl.kernel(
        body,
        mesh=mesh,
        out_type=jax.ShapeDtypeStruct((bsz * seq_len, d), x.dtype),
        scratch_types=[
            pltpu.VMEM((_CH, d), jnp.float32),
            pltpu.VMEM((_CH, d), jnp.float32),
        ],
    )(x2, table)
    return out.reshape(bsz, seq_len, d)


# speedup vs baseline: 1.7878x; 1.7878x over previous
import functools

import jax
import jax.numpy as jnp
from jax import lax
from jax.experimental import pallas as pl
from jax.experimental.pallas import tpu as pltpu
from jax.experimental.pallas import tpu_sc as plsc

MAX_LEN = 8192

_NC = 2
_NS = 16
_NW = _NC * _NS
_CH = 16


def _sc_body(bsz, seq_len, d, x_hbm, t_hbm, o_hbm,
             xv0, xv1, ov0, ov1, tv, in_sem, out_sem):
    wid = lax.axis_index("c") * _NS + lax.axis_index("s")
    seq_per_w = seq_len // _NW
    seq_base = wid * seq_per_w
    groups = d // 16
    n_chunks = (seq_per_w // _CH) * bsz

    def row_of(n):
        g = n // bsz
        b = lax.rem(n, bsz)
        return b * seq_len + seq_base + g * _CH

    def in_copy(n, xv):
        return pltpu.make_async_copy(
            x_hbm.at[pl.ds(row_of(n), _CH), :], xv, in_sem.at[lax.rem(n, 2)])

    def out_copy(n, ov):
        return pltpu.make_async_copy(
            ov, o_hbm.at[pl.ds(row_of(n), _CH), :], out_sem.at[lax.rem(n, 2)])

    in_copy(0, xv0).start()

    def step(n, carry):
        cur = lax.rem(n, 2)

        @pl.when(lax.rem(n, bsz) == 0)
        def _():
            t0 = seq_base + (n // bsz) * _CH
            pltpu.sync_copy(t_hbm.at[pl.ds(t0, _CH), :], tv)

        def with_bufs(xv, ov):
            in_copy(n, xv).wait()

            @pl.when(n + 1 < n_chunks)
            def _():
                def start_next(nxv, nov):
                    in_copy(n + 1, nxv).start()
                lax.cond(cur == 0,
                         lambda: start_next(xv1, ov1),
                         lambda: start_next(xv0, ov0))

            @pl.when(n >= 2)
            def _():
                out_copy(n - 2, ov).wait()

            def row(i, c2):
                for j in range(groups):
                    sl = pl.ds(j * 16, 16)
                    ov[i, sl] = xv[i, sl] + tv[i, sl]
                return c2

            lax.fori_loop(0, _CH, row, 0)
            out_copy(n, ov).start()

        lax.cond(cur == 0,
                 lambda: with_bufs(xv0, ov0),
                 lambda: with_bufs(xv1, ov1))
        return carry

    lax.fori_loop(0, n_chunks, step, 0)

    if n_chunks >= 2:
        out_copy(n_chunks - 2, ov0 if (n_chunks - 2) % 2 == 0 else ov1).wait()
    out_copy(n_chunks - 1, ov0 if (n_chunks - 1) % 2 == 0 else ov1).wait()


def kernel(x, table):
    bsz, seq_len, d = x.shape
    if seq_len > MAX_LEN:
        x = x[:, -MAX_LEN:, :]
        seq_len = MAX_LEN
    x2 = x.reshape(bsz * seq_len, d)
    mesh = plsc.VectorSubcoreMesh(core_axis_name="c", subcore_axis_name="s")
    body = functools.partial(_sc_body, bsz, seq_len, d)
    out = pl.kernel(
        body,
        mesh=mesh,
        out_type=jax.ShapeDtypeStruct((bsz * seq_len, d), x.dtype),
        scratch_types=[
            pltpu.VMEM((_CH, d), jnp.float32),
            pltpu.VMEM((_CH, d), jnp.float32),
            pltpu.VMEM((_CH, d), jnp.float32),
            pltpu.VMEM((_CH, d), jnp.float32),
            pltpu.VMEM((_CH, d), jnp.float32),
            pltpu.SemaphoreType.DMA((2,)),
            pltpu.SemaphoreType.DMA((2,)),
        ],
    )(x2, table)
    return out.reshape(bsz, seq_len, d)


# speedup vs baseline: 3.1024x; 1.7353x over previous
import jax
import jax.numpy as jnp
from jax.experimental import pallas as pl
from jax.experimental.pallas import tpu as pltpu

MAX_LEN = 8192


def _add_kernel(x_ref, t_ref, o_ref):
    o_ref[...] = x_ref[...] + t_ref[...]


def kernel(x, table):
    bsz, seq_len, d = x.shape
    if seq_len > MAX_LEN:
        x = x[:, -MAX_LEN:, :]
        seq_len = MAX_LEN
    bs = 512
    grid = (seq_len // bs,)
    return pl.pallas_call(
        _add_kernel,
        grid=grid,
        in_specs=[
            pl.BlockSpec((bsz, bs, d), lambda j: (0, j, 0)),
            pl.BlockSpec((bs, d), lambda j: (j, 0)),
        ],
        out_specs=pl.BlockSpec((bsz, bs, d), lambda j: (0, j, 0)),
        out_shape=jax.ShapeDtypeStruct(x.shape, x.dtype),
        compiler_params=pltpu.CompilerParams(vmem_limit_bytes=60 * 1024 * 1024),
    )(x, table)
